# baseline (speedup 1.0000x reference)
import jax
import jax.numpy as jnp
from jax.experimental import pallas as pl
from jax.experimental.pallas import tpu as pltpu


def _moe_body(x_ref, gw_ref, gb_ref, eb_ref, ew_hbm, out_ref,
              wbuf_ref, ub_ref, xbn_ref, sem_ref):
    c = pl.program_id(0)
    n_pairs = pl.num_programs(0)
    n_experts = ub_ref.shape[1]

    def w_copy(pair, slot):
        return pltpu.make_async_copy(
            ew_hbm.at[pl.ds(2 * pair, 2)], wbuf_ref.at[slot],
            sem_ref.at[slot])

    def w_single(e):
        return pltpu.make_async_copy(
            ew_hbm.at[e], wbuf_ref.at[0, e], sem_ref.at[2 + e])

    @pl.when(c == 0)
    def _init():
        w_single(0).start()
        w_single(1).start()
        w_copy(1, 1).start()
        xb = x_ref[...].astype(jnp.bfloat16)
        gwb = gw_ref[...].astype(jnp.bfloat16)
        logits = jax.lax.dot_general(
            xb, gwb, (((1,), (1,)), ((), ())),
            preferred_element_type=jnp.float32)
        logits = logits + gb_ref[...]
        u = jnp.exp(logits)
        z = jnp.sum(u, axis=-1, keepdims=True)
        rz = 1.0 / z
        xbn_ref[...] = xb * rz.astype(jnp.bfloat16)
        ub_ref[...] = u.astype(jnp.bfloat16)
        probs = u * rz
        out_ref[...] = jax.lax.dot_general(
            probs.astype(jnp.bfloat16), eb_ref[...].astype(jnp.bfloat16),
            (((1,), (0,)), ((), ())), preferred_element_type=jnp.float32)

    slot = jax.lax.rem(c, 2)

    @pl.when(c == 0)
    def _wait_first():
        w_single(0).wait()

    @pl.when(c != 0)
    def _wait_pair():
        w_copy(c, slot).wait()

    lane = jax.lax.broadcasted_iota(jnp.int32, (1, n_experts), 1)
    u_all = ub_ref[...]
    e0 = 2 * c
    u0 = jnp.sum(jnp.where(lane == e0, u_all.astype(jnp.float32), 0.0),
                 axis=1, keepdims=True).astype(jnp.bfloat16)
    u1 = jnp.sum(jnp.where(lane == e0 + 1, u_all.astype(jnp.float32), 0.0),
                 axis=1, keepdims=True).astype(jnp.bfloat16)
    xbn = xbn_ref[...]
    xs0 = xbn * u0
    xs1 = xbn * u1
    wb0 = wbuf_ref[slot, 0].astype(jnp.bfloat16)
    y = jax.lax.dot_general(
        xs0, wb0, (((1,), (1,)), ((), ())),
        preferred_element_type=jnp.float32)

    @pl.when(c == 0)
    def _wait_second():
        w_single(1).wait()

    wb1 = wbuf_ref[slot, 1].astype(jnp.bfloat16)
    y = y + jax.lax.dot_general(
        xs1, wb1, (((1,), (1,)), ((), ())),
        preferred_element_type=jnp.float32)
    out_ref[...] += y

    @pl.when(c + 2 < n_pairs)
    def _prefetch():
        w_copy(c + 2, slot).start()


def kernel(x, gate_w, gate_b, expert_w, expert_b):
    b, s, d = x.shape
    n_e, h, _ = expert_w.shape
    t = b * s
    x_flat = x.reshape(t, d)
    out = pl.pallas_call(
        _moe_body,
        grid=(n_e // 2,),
        in_specs=[
            pl.BlockSpec((t, d), lambda c: (0, 0)),
            pl.BlockSpec((n_e, d), lambda c: (0, 0)),
            pl.BlockSpec((1, n_e), lambda c: (0, 0)),
            pl.BlockSpec((n_e, h), lambda c: (0, 0)),
            pl.BlockSpec(memory_space=pltpu.MemorySpace.HBM),
        ],
        out_specs=pl.BlockSpec((t, h), lambda c: (0, 0)),
        out_shape=jax.ShapeDtypeStruct((t, h), jnp.float32),
        scratch_shapes=[
            pltpu.VMEM((2, 2, h, d), jnp.float32),
            pltpu.VMEM((t, n_e), jnp.bfloat16),
            pltpu.VMEM((t, d), jnp.bfloat16),
            pltpu.SemaphoreType.DMA((4,)),
        ],
        compiler_params=pltpu.CompilerParams(
            dimension_semantics=("arbitrary",)),
    )(x_flat, gate_w, gate_b.reshape(1, n_e), expert_b, expert_w)
    return out.reshape(b, s, h)


# speedup vs baseline: 1.0749x; 1.0749x over previous
import jax
import jax.numpy as jnp
from jax.experimental import pallas as pl
from jax.experimental.pallas import tpu as pltpu


def _moe_body(x_ref, gw_ref, gb_ref, eb_ref, ew_hbm, out_ref,
              wbuf_ref, ub_ref, xbn_ref, sem_ref):
    c = pl.program_id(0)
    n_pairs = pl.num_programs(0)
    n_experts = ub_ref.shape[1]

    def w_copy(pair, slot):
        return pltpu.make_async_copy(
            ew_hbm.at[pl.ds(2 * pair, 2)], wbuf_ref.at[slot],
            sem_ref.at[slot])

    @pl.when(c == 0)
    def _init():
        w_copy(0, 0).start()
        w_copy(1, 1).start()
        xb = x_ref[...].astype(jnp.bfloat16)
        gwb = gw_ref[...].astype(jnp.bfloat16)
        logits = jax.lax.dot_general(
            xb, gwb, (((1,), (1,)), ((), ())),
            preferred_element_type=jnp.float32)
        logits = logits + gb_ref[...]
        u = jnp.exp(logits)
        z = jnp.sum(u, axis=-1, keepdims=True)
        rz = 1.0 / z
        xbn_ref[...] = xb * rz.astype(jnp.bfloat16)
        ub_ref[...] = u.astype(jnp.bfloat16)
        probs = u * rz
        out_ref[...] = jax.lax.dot_general(
            probs.astype(jnp.bfloat16), eb_ref[...].astype(jnp.bfloat16),
            (((1,), (0,)), ((), ())), preferred_element_type=jnp.float32)

    slot = jax.lax.rem(c, 2)
    w_copy(c, slot).wait()

    lane = jax.lax.broadcasted_iota(jnp.int32, (1, n_experts), 1)
    u_all = ub_ref[...]
    e0 = 2 * c
    u0 = jnp.sum(jnp.where(lane == e0, u_all.astype(jnp.float32), 0.0),
                 axis=1, keepdims=True).astype(jnp.bfloat16)
    u1 = jnp.sum(jnp.where(lane == e0 + 1, u_all.astype(jnp.float32), 0.0),
                 axis=1, keepdims=True).astype(jnp.bfloat16)
    xbn = xbn_ref[...]
    xs0 = xbn * u0
    xs1 = xbn * u1
    wb0 = wbuf_ref[slot, 0].astype(jnp.bfloat16)
    wb1 = wbuf_ref[slot, 1].astype(jnp.bfloat16)
    y = jax.lax.dot_general(
        xs0, wb0, (((1,), (1,)), ((), ())),
        preferred_element_type=jnp.float32)
    y = y + jax.lax.dot_general(
        xs1, wb1, (((1,), (1,)), ((), ())),
        preferred_element_type=jnp.float32)
    out_ref[...] += y

    @pl.when(c + 2 < n_pairs)
    def _prefetch():
        w_copy(c + 2, slot).start()


def kernel(x, gate_w, gate_b, expert_w, expert_b):
    b, s, d = x.shape
    n_e, h, _ = expert_w.shape
    t = b * s
    x_flat = x.reshape(t, d)
    out = pl.pallas_call(
        _moe_body,
        grid=(n_e // 2,),
        in_specs=[
            pl.BlockSpec((t, d), lambda c: (0, 0)),
            pl.BlockSpec((n_e, d), lambda c: (0, 0)),
            pl.BlockSpec((1, n_e), lambda c: (0, 0)),
            pl.BlockSpec((n_e, h), lambda c: (0, 0)),
            pl.BlockSpec(memory_space=pltpu.MemorySpace.HBM),
        ],
        out_specs=pl.BlockSpec((t, h), lambda c: (0, 0)),
        out_shape=jax.ShapeDtypeStruct((t, h), jnp.float32),
        scratch_shapes=[
            pltpu.VMEM((2, 2, h, d), jnp.float32),
            pltpu.VMEM((t, n_e), jnp.bfloat16),
            pltpu.VMEM((t, d), jnp.bfloat16),
            pltpu.SemaphoreType.DMA((2,)),
        ],
        compiler_params=pltpu.CompilerParams(
            dimension_semantics=("arbitrary",)),
    )(x_flat, gate_w, gate_b.reshape(1, n_e), expert_b, expert_w)
    return out.reshape(b, s, h)
